# baseline (speedup 1.0000x reference)
import jax
import jax.numpy as jnp
from jax import lax
from jax.experimental import pallas as pl
from jax.experimental.pallas import tpu as pltpu
from jax.experimental.pallas import tpu_sc as plsc

_NC, _NS = 2, 16
_NW = _NC * _NS


def _sc_gather_body(table_hbm, idx_hbm, out_hbm, idx_v, rows_v, sem):
    bpw = idx_v.shape[0]
    wid = lax.axis_index("s") * _NC + lax.axis_index("c")
    base = wid * bpw
    pltpu.sync_copy(idx_hbm.at[pl.ds(base, bpw)], idx_v)
    pltpu.async_copy(table_hbm.at[idx_v], rows_v, sem).wait()
    pltpu.sync_copy(rows_v, out_hbm.at[pl.ds(base, bpw)])


def _mm_body(head_ref, g_ref, wt_ref, bt_ref, wg_ref, bg_ref,
             tok_ref, graph_ref):
    g = g_ref[...]
    t = lax.dot_general(g, wt_ref[...], (((1,), (1,)), ((), ())),
                        preferred_element_type=jnp.float32)
    t = t + bt_ref[...][None, :]
    h = lax.dot_general(g, wg_ref[...], (((1,), (1,)), ((), ())),
                        preferred_element_type=jnp.float32)
    h = h + bg_ref[...][None, :]
    tok_ref[:, 0:1, :] = t[:, None, :]
    tok_ref[:, 1:, :] = head_ref[:, 1:, :]
    graph_ref[...] = h


def _scatter_body(idx_ref, g_ref, auf_ref, out_ref, sem):
    del auf_ref
    n = g_ref.shape[0]

    def fire(i, _):
        pltpu.make_async_copy(
            g_ref.at[pl.ds(i, 1)],
            out_ref.at[pl.ds(idx_ref[i], 1)],
            sem,
        ).start()
        return 0

    def drain(i, _):
        pltpu.make_async_copy(
            g_ref.at[pl.ds(i, 1)],
            out_ref.at[pl.ds(idx_ref[i], 1)],
            sem,
        ).wait()
        return 0

    lax.fori_loop(0, n, fire, 0)
    lax.fori_loop(0, n, drain, 0)


def kernel(text, all_user_feature, user_neighbor_index,
           W_text, b_text, W_graph, b_graph):
    B, S, D = text.shape
    M = all_user_feature.shape[0]
    user_index = user_neighbor_index[:, 0]

    bpw = B // _NW
    graph_ini = pl.kernel(
        _sc_gather_body,
        out_type=jax.ShapeDtypeStruct((B, D), jnp.float32),
        mesh=plsc.VectorSubcoreMesh(core_axis_name="c", subcore_axis_name="s"),
        scratch_types=[
            pltpu.VMEM((bpw,), jnp.int32),
            pltpu.VMEM((bpw, D), jnp.float32),
            pltpu.SemaphoreType.DMA,
        ],
    )(all_user_feature, user_index)

    text_out, graph = pl.pallas_call(
        _mm_body,
        grid=(1,),
        in_specs=[
            pl.BlockSpec((B, 8, D), lambda i: (0, 0, 0)),
            pl.BlockSpec((B, D), lambda i: (0, 0)),
            pl.BlockSpec((D, D), lambda i: (0, 0)),
            pl.BlockSpec((D,), lambda i: (0,)),
            pl.BlockSpec((D, D), lambda i: (0, 0)),
            pl.BlockSpec((D,), lambda i: (0,)),
        ],
        out_specs=[
            pl.BlockSpec((B, 8, D), lambda i: (0, 0, 0)),
            pl.BlockSpec((B, D), lambda i: (0, 0)),
        ],
        out_shape=[
            jax.ShapeDtypeStruct((B, S, D), jnp.float32),
            jax.ShapeDtypeStruct((B, D), jnp.float32),
        ],
        input_output_aliases={0: 0},
    )(text, graph_ini, W_text, b_text, W_graph, b_graph)

    new_auf = pl.pallas_call(
        _scatter_body,
        in_specs=[
            pl.BlockSpec(memory_space=pltpu.SMEM),
            pl.BlockSpec((B, D), lambda: (0, 0)),
            pl.BlockSpec(memory_space=pl.ANY),
        ],
        out_specs=pl.BlockSpec(memory_space=pl.ANY),
        out_shape=jax.ShapeDtypeStruct((M, D), jnp.float32),
        scratch_shapes=[pltpu.SemaphoreType.DMA],
        input_output_aliases={2: 0},
    )(user_index, graph, all_user_feature)

    return (text_out, new_auf)
